# Initial kernel scaffold; baseline (speedup 1.0000x reference)
import jax
import jax.numpy as jnp
from jax.experimental import pallas as pl


def kernel(h, w, edge_index, W, a):
    raise NotImplementedError("write your pallas kernel here")


# speedup vs baseline: 21.9446x; 21.9446x over previous
import functools

import jax
import jax.numpy as jnp
from jax import lax
from jax.experimental import pallas as pl
from jax.experimental.pallas import tpu as pltpu
from jax.experimental.pallas import tpu_sc as plsc

NC = 2
NS = 16
NW = NC * NS
L = 16
CHUNK = 128


def _lane_bcast(vec, t):
    idx = jnp.full((L, 1), t, jnp.int32)
    return lax.gather(
        vec, idx,
        lax.GatherDimensionNumbers(offset_dims=(), collapsed_slice_dims=(0,),
                                   start_index_map=(0,)),
        slice_sizes=(1,), mode=lax.GatherScatterMode.PROMISE_IN_BOUNDS)


def _prep_body(h_ref, w_ref, a_ref, z_ref, s1_ref, s2_ref):
    z = lax.dot_general(h_ref[...], w_ref[...], (((1,), (1,)), ((), ())),
                        preferred_element_type=jnp.float32)
    z_ref[...] = z
    s1_ref[...] = jnp.sum(z * a_ref[0:1, :], axis=1)
    s2_ref[...] = jnp.sum(z * a_ref[1:2, :], axis=1)


def _combine_body(p_ref, d_ref, o_ref):
    den = jnp.maximum(jnp.sum(d_ref[...], axis=0), 1e-16)
    o_ref[...] = (p_ref[0] + p_ref[1]) / den[:, None]


def _sc_body(n_nodes, n_chunks, chunks_per_tile,
             z_hbm, src_hbm, dst_hbm, s1_hbm, s2_hbm,
             out_p_hbm, out_d_hbm,
             s1_v, s2_v, den_v, si_v, di_v, rows_v, acc_sh, sem):
    cid = lax.axis_index("c")
    sid = lax.axis_index("s")
    wid = sid * NC + cid
    rps = (n_nodes // (NS * 8)) * 8
    tail = n_nodes - NS * rps

    pltpu.sync_copy(s1_hbm, s1_v)
    pltpu.sync_copy(s2_hbm, s2_v)

    def _zero_den(i, _):
        den_v[pl.ds(pl.multiple_of(i * L, L), L)] = jnp.zeros((L,), jnp.float32)
        return 0
    lax.fori_loop(0, n_nodes // L, _zero_den, 0)

    def _zero_rows(i, _):
        for q in range(8):
            rows_v[i, pl.ds(q * L, L)] = jnp.zeros((L,), jnp.float32)
        return 0
    lax.fori_loop(0, CHUNK, _zero_rows, 0)
    def _zero_acc_rows(r0, nrows):
        nzc = nrows // CHUNK
        for b in range(nzc):
            pltpu.sync_copy(rows_v, acc_sh.at[pl.ds(r0 + b * CHUNK, CHUNK)])
        rem = nrows - nzc * CHUNK
        if rem:
            pltpu.sync_copy(rows_v.at[pl.ds(0, rem)],
                            acc_sh.at[pl.ds(r0 + nzc * CHUNK, rem)])

    _zero_acc_rows(pl.multiple_of(sid * rps, 8), rps)
    if tail:
        @pl.when(sid == 0)
        def _():
            _zero_acc_rows(NS * rps, tail)
    plsc.subcore_barrier()

    def _chunk(i, _):
        c = i * NW + wid

        @pl.when(c < n_chunks)
        def _():
            base = pl.multiple_of(c * CHUNK, CHUNK)
            pltpu.sync_copy(src_hbm.at[pl.ds(base, CHUNK)], si_v)
            pltpu.sync_copy(dst_hbm.at[pl.ds(base, CHUNK)], di_v)
            pltpu.async_copy(z_hbm.at[si_v], rows_v, sem).wait()
            for v in range(CHUNK // L):
                sv = si_v[pl.ds(v * L, L)]
                dv = di_v[pl.ds(v * L, L)]
                e = plsc.load_gather(s1_v, [sv]) + plsc.load_gather(s2_v, [dv])
                e = jnp.where(e >= 0.0, e, 0.01 * e)
                ee = jnp.exp(e)
                plsc.addupdate_scatter(den_v, [dv], ee)
                for t in range(L):
                    s = _lane_bcast(ee, t)
                    j = v * L + t
                    for q in range(8):
                        rows_v[j, pl.ds(q * L, L)] = (
                            rows_v[j, pl.ds(q * L, L)] * s)
            pltpu.sync_copy(rows_v, acc_sh.at[di_v], add=True)
        return 0

    lax.fori_loop(0, chunks_per_tile, _chunk, 0)
    plsc.subcore_barrier()

    pltpu.sync_copy(den_v, out_d_hbm.at[wid, 0])
    r0 = pl.multiple_of(sid * rps, 8)
    pltpu.sync_copy(acc_sh.at[pl.ds(r0, rps)],
                    out_p_hbm.at[cid, pl.ds(r0, rps)])
    if tail:
        @pl.when(sid == 0)
        def _():
            pltpu.sync_copy(acc_sh.at[pl.ds(NS * rps, tail)],
                            out_p_hbm.at[cid, pl.ds(NS * rps, tail)])


def kernel(h, w, edge_index, W, a):
    n, d_in = h.shape
    d_out = W.shape[0]
    e_cnt = edge_index.shape[1]
    n_chunks = e_cnt // CHUNK
    chunks_per_tile = (n_chunks + NW - 1) // NW

    z, s1, s2 = pl.pallas_call(
        _prep_body,
        out_shape=[
            jax.ShapeDtypeStruct((n, d_out), jnp.float32),
            jax.ShapeDtypeStruct((n,), jnp.float32),
            jax.ShapeDtypeStruct((n,), jnp.float32),
        ],
    )(h, W, a.reshape(2, d_out))

    src = edge_index[0]
    dst = edge_index[1]

    mesh = plsc.VectorSubcoreMesh(core_axis_name="c", subcore_axis_name="s",
                                  num_cores=NC, num_subcores=NS)
    sc_fn = functools.partial(
        pl.kernel,
        out_type=[
            jax.ShapeDtypeStruct((NC, n, d_out), jnp.float32),
            jax.ShapeDtypeStruct((NW, 1, n), jnp.float32),
        ],
        mesh=mesh,
        scratch_types=[
            pltpu.VMEM((n,), jnp.float32),
            pltpu.VMEM((n,), jnp.float32),
            pltpu.VMEM((n,), jnp.float32),
            pltpu.VMEM((CHUNK,), jnp.int32),
            pltpu.VMEM((CHUNK,), jnp.int32),
            pltpu.VMEM((CHUNK, d_out), jnp.float32),
            pltpu.VMEM_SHARED((n, d_out), jnp.float32),
            pltpu.SemaphoreType.DMA,
        ],
        compiler_params=pltpu.CompilerParams(needs_layout_passes=False),
    )(functools.partial(_sc_body, n, n_chunks, chunks_per_tile))
    partials, denoms = sc_fn(z, src, dst, s1, s2)

    h_out = pl.pallas_call(
        _combine_body,
        out_shape=jax.ShapeDtypeStruct((n, d_out), jnp.float32),
    )(partials, denoms.reshape(NW, n))
    return h_out


# speedup vs baseline: 35.6600x; 1.6250x over previous
import functools

import jax
import jax.numpy as jnp
from jax import lax
from jax.experimental import pallas as pl
from jax.experimental.pallas import tpu as pltpu
from jax.experimental.pallas import tpu_sc as plsc

NC = 2
NS = 16
NW = NC * NS
L = 16
CHUNK = 64


def _lane_bcast(vec, t):
    idx = jnp.full((L, 1), t, jnp.int32)
    return lax.gather(
        vec, idx,
        lax.GatherDimensionNumbers(offset_dims=(), collapsed_slice_dims=(0,),
                                   start_index_map=(0,)),
        slice_sizes=(1,), mode=lax.GatherScatterMode.PROMISE_IN_BOUNDS)


def _prep_body(h_ref, w_ref, a_ref, z_ref, s1_ref, s2_ref):
    z = lax.dot_general(h_ref[...], w_ref[...], (((1,), (1,)), ((), ())),
                        preferred_element_type=jnp.float32)
    z_ref[...] = z
    s1_ref[...] = jnp.sum(z * a_ref[0:1, :], axis=1)
    s2_ref[...] = jnp.sum(z * a_ref[1:2, :], axis=1)


def _combine_body(p_ref, d_ref, o_ref):
    den = jnp.maximum(jnp.sum(d_ref[...], axis=0), 1e-16)
    o_ref[...] = (p_ref[0] + p_ref[1]) / den[:, None]


def _sc_body(n_nodes, n_chunks, chunks_per_tile,
             z_hbm, src_hbm, dst_hbm, s1_hbm, s2_hbm,
             out_p_hbm, out_d_hbm,
             s1_v, s2_v, den_v, si_v, di_v, dsc_v, rows_v, acc_sh,
             sem_i, sem_g, sem_s):
    cid = lax.axis_index("c")
    sid = lax.axis_index("s")
    wid = sid * NC + cid
    rps = (n_nodes // (NS * 8)) * 8
    tail = n_nodes - NS * rps

    pltpu.sync_copy(s1_hbm, s1_v)
    pltpu.sync_copy(s2_hbm, s2_v)

    def _zero_den(i, _):
        den_v[pl.ds(pl.multiple_of(i * L, L), L)] = jnp.zeros((L,), jnp.float32)
        return 0
    lax.fori_loop(0, n_nodes // L, _zero_den, 0)

    def _zero_rows(i, _):
        for q in range(8):
            rows_v[0][i, pl.ds(q * L, L)] = jnp.zeros((L,), jnp.float32)
        return 0
    lax.fori_loop(0, CHUNK, _zero_rows, 0)
    def _zero_acc_rows(r0, nrows):
        nzc = nrows // CHUNK
        for b in range(nzc):
            pltpu.sync_copy(rows_v[0],
                            acc_sh.at[pl.ds(r0 + b * CHUNK, CHUNK)])
        rem = nrows - nzc * CHUNK
        if rem:
            pltpu.sync_copy(rows_v[0].at[pl.ds(0, rem)],
                            acc_sh.at[pl.ds(r0 + nzc * CHUNK, rem)])

    _zero_acc_rows(pl.multiple_of(sid * rps, 8), rps)
    if tail:
        @pl.when(sid == 0)
        def _():
            _zero_acc_rows(NS * rps, tail)
    plsc.subcore_barrier()

    cnt = (n_chunks - wid + NW - 1) // NW

    def _base(i):
        return pl.multiple_of((i * NW + wid) * CHUNK, CHUNK)

    def _idx_issue(i, b):
        pltpu.async_copy(src_hbm.at[pl.ds(_base(i), CHUNK)], si_v[b],
                         sem_i[b])
        pltpu.async_copy(dst_hbm.at[pl.ds(_base(i), CHUNK)], di_v[b],
                         sem_i[b])

    def _idx_wait(i, b):
        pltpu.make_async_copy(src_hbm.at[pl.ds(_base(i), CHUNK)], si_v[b],
                              sem_i[b]).wait()
        pltpu.make_async_copy(dst_hbm.at[pl.ds(_base(i), CHUNK)], di_v[b],
                              sem_i[b]).wait()

    def _gather_issue(b):
        pltpu.async_copy(z_hbm.at[si_v[b]], rows_v[b], sem_g[b])

    def _gather_wait(b):
        pltpu.make_async_copy(z_hbm.at[si_v[b]], rows_v[b], sem_g[b]).wait()

    def _scatter_wait(b):
        pltpu.make_async_copy(rows_v[b], acc_sh.at[dsc_v[b]],
                              sem_s[b]).wait()

    @pl.when(cnt >= 1)
    def _():
        _idx_issue(0, 0)

    @pl.when(cnt >= 2)
    def _():
        _idx_issue(1, 1)

    @pl.when(cnt >= 1)
    def _():
        _idx_wait(0, 0)
        _gather_issue(0)

    def _process(i, b):
        b1 = 1 - b

        @pl.when(jnp.logical_and(i + 1 < cnt, i >= 1))
        def _():
            _scatter_wait(b1)

        @pl.when(i + 1 < cnt)
        def _():
            _idx_wait(i + 1, b1)
            _gather_issue(b1)

        @pl.when(i < cnt)
        def _():
            _gather_wait(b)

            def _group(v, _):
                o = pl.multiple_of(v * L, L)
                sv = si_v[b][pl.ds(o, L)]
                dv = di_v[b][pl.ds(o, L)]
                e = (plsc.load_gather(s1_v, [sv])
                     + plsc.load_gather(s2_v, [dv]))
                e = jnp.where(e >= 0.0, e, 0.01 * e)
                ee = jnp.exp(e)
                plsc.addupdate_scatter(den_v, [dv], ee)
                dsc_v[b][pl.ds(o, L)] = dv
                for t in range(L):
                    s = _lane_bcast(ee, t)
                    j = v * L + t
                    for q in range(8):
                        rows_v[b][j, pl.ds(q * L, L)] = (
                            rows_v[b][j, pl.ds(q * L, L)] * s)
                return 0

            lax.fori_loop(0, CHUNK // L, _group, 0)
            pltpu.async_copy(rows_v[b], acc_sh.at[dsc_v[b]], sem_s[b],
                             add=True)

        @pl.when(i + 2 < cnt)
        def _():
            _idx_issue(i + 2, b)

    def _pair(k, _):
        _process(2 * k, 0)
        _process(2 * k + 1, 1)
        return 0

    lax.fori_loop(0, (chunks_per_tile + 1) // 2, _pair, 0)
    _scatter_wait(0)
    _scatter_wait(1)
    plsc.subcore_barrier()

    pltpu.sync_copy(den_v, out_d_hbm.at[wid, 0])
    r0 = pl.multiple_of(sid * rps, 8)
    pltpu.sync_copy(acc_sh.at[pl.ds(r0, rps)],
                    out_p_hbm.at[cid, pl.ds(r0, rps)])
    if tail:
        @pl.when(sid == 0)
        def _():
            pltpu.sync_copy(acc_sh.at[pl.ds(NS * rps, tail)],
                            out_p_hbm.at[cid, pl.ds(NS * rps, tail)])


def kernel(h, w, edge_index, W, a):
    n, d_in = h.shape
    d_out = W.shape[0]
    e_cnt = edge_index.shape[1]
    n_chunks = e_cnt // CHUNK
    chunks_per_tile = (n_chunks + NW - 1) // NW

    z, s1, s2 = pl.pallas_call(
        _prep_body,
        out_shape=[
            jax.ShapeDtypeStruct((n, d_out), jnp.float32),
            jax.ShapeDtypeStruct((n,), jnp.float32),
            jax.ShapeDtypeStruct((n,), jnp.float32),
        ],
    )(h, W, a.reshape(2, d_out))

    src = edge_index[0]
    dst = edge_index[1]

    mesh = plsc.VectorSubcoreMesh(core_axis_name="c", subcore_axis_name="s",
                                  num_cores=NC, num_subcores=NS)
    sc_fn = functools.partial(
        pl.kernel,
        out_type=[
            jax.ShapeDtypeStruct((NC, n, d_out), jnp.float32),
            jax.ShapeDtypeStruct((NW, 1, n), jnp.float32),
        ],
        mesh=mesh,
        scratch_types=[
            pltpu.VMEM((n,), jnp.float32),
            pltpu.VMEM((n,), jnp.float32),
            pltpu.VMEM((n,), jnp.float32),
            [pltpu.VMEM((CHUNK,), jnp.int32) for _ in range(2)],
            [pltpu.VMEM((CHUNK,), jnp.int32) for _ in range(2)],
            [pltpu.VMEM((CHUNK,), jnp.int32) for _ in range(2)],
            [pltpu.VMEM((CHUNK, d_out), jnp.float32) for _ in range(2)],
            pltpu.VMEM_SHARED((n, d_out), jnp.float32),
            [pltpu.SemaphoreType.DMA for _ in range(2)],
            [pltpu.SemaphoreType.DMA for _ in range(2)],
            [pltpu.SemaphoreType.DMA for _ in range(2)],
        ],
        compiler_params=pltpu.CompilerParams(needs_layout_passes=False),
    )(functools.partial(_sc_body, n, n_chunks, chunks_per_tile))
    partials, denoms = sc_fn(z, src, dst, s1, s2)

    h_out = pl.pallas_call(
        _combine_body,
        out_shape=jax.ShapeDtypeStruct((n, d_out), jnp.float32),
    )(partials, denoms.reshape(NW, n))
    return h_out


# speedup vs baseline: 37.9263x; 1.0636x over previous
import functools

import jax
import jax.numpy as jnp
from jax import lax
from jax.experimental import pallas as pl
from jax.experimental.pallas import tpu as pltpu
from jax.experimental.pallas import tpu_sc as plsc

NC = 2
NS = 16
NW = NC * NS
L = 16
CHUNK = 48
NB = 6
NR = 3


def _lane_bcast(vec, t):
    idx = jnp.full((L, 1), t, jnp.int32)
    return lax.gather(
        vec, idx,
        lax.GatherDimensionNumbers(offset_dims=(), collapsed_slice_dims=(0,),
                                   start_index_map=(0,)),
        slice_sizes=(1,), mode=lax.GatherScatterMode.PROMISE_IN_BOUNDS)


def _prep_body(h_ref, w_ref, a_ref, z_ref, s1_ref, s2_ref):
    z = lax.dot_general(h_ref[...], w_ref[...], (((1,), (1,)), ((), ())),
                        preferred_element_type=jnp.float32)
    z_ref[...] = z
    s1_ref[...] = jnp.sum(z * a_ref[0:1, :], axis=1)
    s2_ref[...] = jnp.sum(z * a_ref[1:2, :], axis=1)


def _combine_body(p_ref, d_ref, o_ref):
    den = jnp.maximum(jnp.sum(d_ref[...], axis=0), 1e-16)
    o_ref[...] = (p_ref[0] + p_ref[1]) / den[:, None]


def _sc_body(n_nodes, n_chunks, chunks_per_tile, e_cnt,
             z_hbm, src_hbm, dst_hbm, s1_hbm, s2_hbm,
             out_p_hbm, out_d_hbm,
             s1_v, s2_v, den_v, si_v, di_v, dsc_v, rows_v, acc_sh,
             sem_i, sem_g, sem_s):
    cid = lax.axis_index("c")
    sid = lax.axis_index("s")
    wid = sid * NC + cid
    rps = (n_nodes // (NS * 8)) * 8
    tail = n_nodes - NS * rps

    pltpu.sync_copy(s1_hbm, s1_v)
    pltpu.sync_copy(s2_hbm, s2_v)

    def _zero_den(i, _):
        den_v[pl.ds(pl.multiple_of(i * L, L), L)] = jnp.zeros((L,), jnp.float32)
        return 0
    lax.fori_loop(0, n_nodes // L, _zero_den, 0)

    def _zero_rows(i, _):
        for qz in range(8):
            rows_v[0][i, pl.ds(qz * L, L)] = jnp.zeros((L,), jnp.float32)
        return 0
    lax.fori_loop(0, CHUNK, _zero_rows, 0)

    def _zero_acc_rows(r0, nrows):
        nzc = nrows // CHUNK
        for bz in range(nzc):
            pltpu.sync_copy(rows_v[0],
                            acc_sh.at[pl.ds(r0 + bz * CHUNK, CHUNK)])
        rem = nrows - nzc * CHUNK
        if rem:
            pltpu.sync_copy(rows_v[0].at[pl.ds(0, rem)],
                            acc_sh.at[pl.ds(r0 + nzc * CHUNK, rem)])

    _zero_acc_rows(pl.multiple_of(sid * rps, 8), rps)
    if tail:
        @pl.when(sid == 0)
        def _():
            _zero_acc_rows(NS * rps, tail)
    plsc.subcore_barrier()

    q, r = divmod(n_chunks, NW)
    cnt = q + (wid < r).astype(jnp.int32)
    start = wid * q + jnp.minimum(wid, r)

    def _blk_slice(j):
        return pl.ds(pl.multiple_of((start + j * NB) * CHUNK, 8),
                     NB * CHUNK)

    def _idx_issue(j, jb):
        pltpu.async_copy(src_hbm.at[_blk_slice(j)], si_v[jb], sem_i[jb])
        pltpu.async_copy(dst_hbm.at[_blk_slice(j)], di_v[jb], sem_i[jb])

    def _idx_wait(j, jb):
        pltpu.make_async_copy(src_hbm.at[_blk_slice(j)], si_v[jb],
                              sem_i[jb]).wait()
        pltpu.make_async_copy(dst_hbm.at[_blk_slice(j)], di_v[jb],
                              sem_i[jb]).wait()

    def _gather_issue(jb, k, b):
        pltpu.async_copy(
            z_hbm.at[si_v[jb].at[pl.ds(k * CHUNK, CHUNK)]], rows_v[b],
            sem_g[b])

    def _gather_wait(jb, k, b):
        pltpu.make_async_copy(
            z_hbm.at[si_v[jb].at[pl.ds(k * CHUNK, CHUNK)]], rows_v[b],
            sem_g[b]).wait()

    def _scatter_wait(b):
        pltpu.make_async_copy(rows_v[b], acc_sh.at[dsc_v[b]],
                              sem_s[b]).wait()

    @pl.when(cnt >= 1)
    def _():
        _idx_issue(0, 0)

    @pl.when(cnt > NB)
    def _():
        _idx_issue(1, 1)

    @pl.when(cnt >= 1)
    def _():
        _idx_wait(0, 0)
        _gather_issue(0, 0, 0)

    @pl.when(cnt >= 2)
    def _():
        _gather_issue(0, 1, 1)

    def _chunkstep(j, jb, k):
        i = j * NB + k
        b = k % NR
        bg = (k + 2) % NR
        kg = (k + 2) % NB
        jbg = jb if k < NB - 2 else 1 - jb

        if k == NB - 2:
            @pl.when(i + 2 < cnt)
            def _():
                _idx_wait(j + 1, jbg)

        @pl.when(i < cnt)
        def _():
            _gather_wait(jb, k, b)

            def _group(v, _):
                o = pl.multiple_of(k * CHUNK + v * L, L)
                od = pl.multiple_of(v * L, L)
                sv = si_v[jb][pl.ds(o, L)]
                dv = di_v[jb][pl.ds(o, L)]
                e = (plsc.load_gather(s1_v, [sv])
                     + plsc.load_gather(s2_v, [dv]))
                e = jnp.where(e >= 0.0, e, 0.01 * e)
                ee = jnp.exp(e)
                plsc.addupdate_scatter(den_v, [dv], ee)
                dsc_v[b][pl.ds(od, L)] = dv
                for t in range(L):
                    s = _lane_bcast(ee, t)
                    jr = v * L + t
                    for qq in range(8):
                        rows_v[b][jr, pl.ds(qq * L, L)] = (
                            rows_v[b][jr, pl.ds(qq * L, L)] * s)
                return 0

            lax.fori_loop(0, CHUNK // L, _group, 0)
            pltpu.async_copy(rows_v[b], acc_sh.at[dsc_v[b]], sem_s[b],
                             add=True)

        @pl.when(jnp.logical_and(i + 2 < cnt, i >= 1))
        def _():
            _scatter_wait(bg)

        @pl.when(i + 2 < cnt)
        def _():
            _gather_issue(jbg, kg, bg)

    def _blockstep(j, jb):
        for k in range(NB):
            _chunkstep(j, jb, k)

        @pl.when((j + 2) * NB < cnt)
        def _():
            _idx_issue(j + 2, jb)

    def _bpair(p, _):
        _blockstep(2 * p, 0)
        _blockstep(2 * p + 1, 1)
        return 0

    nblk_max = (chunks_per_tile + NB - 1) // NB
    lax.fori_loop(0, (nblk_max + 1) // 2, _bpair, 0)
    _scatter_wait(0)
    _scatter_wait(1)
    _scatter_wait(2)

    rem = n_chunks * CHUNK
    nrem = e_cnt - rem

    if nrem:
        @pl.when(wid == NW - 1)
        def _():
            pltpu.sync_copy(src_hbm.at[pl.ds(rem, nrem)],
                            si_v[0].at[pl.ds(0, nrem)])
            pltpu.sync_copy(dst_hbm.at[pl.ds(rem, nrem)],
                            di_v[0].at[pl.ds(0, nrem)])
            pltpu.async_copy(
                z_hbm.at[si_v[0].at[pl.ds(0, nrem)]],
                rows_v[0].at[pl.ds(0, nrem)], sem_g[0]).wait()
            for i in range(CHUNK - nrem):
                for qz in range(8):
                    rows_v[0][nrem + i, pl.ds(qz * L, L)] = (
                        jnp.zeros((L,), jnp.float32))
            for v in range(CHUNK // L):
                od = v * L
                if od < nrem:
                    sv = si_v[0][pl.ds(od, L)]
                    dv = di_v[0][pl.ds(od, L)]
                    e = (plsc.load_gather(s1_v, [sv])
                         + plsc.load_gather(s2_v, [dv]))
                    e = jnp.where(e >= 0.0, e, 0.01 * e)
                    ee = jnp.exp(e)
                    plsc.addupdate_scatter(den_v, [dv], ee)
                    dsc_v[0][pl.ds(od, L)] = dv
                    for t in range(L):
                        s = _lane_bcast(ee, t)
                        for qq in range(8):
                            rows_v[0][od + t, pl.ds(qq * L, L)] = (
                                rows_v[0][od + t, pl.ds(qq * L, L)] * s)
                else:
                    dsc_v[0][pl.ds(od, L)] = jnp.zeros((L,), jnp.int32)
            pltpu.sync_copy(rows_v[0], acc_sh.at[dsc_v[0]], add=True)

    plsc.subcore_barrier()

    pltpu.sync_copy(den_v, out_d_hbm.at[wid, 0])
    r0 = pl.multiple_of(sid * rps, 8)
    pltpu.sync_copy(acc_sh.at[pl.ds(r0, rps)],
                    out_p_hbm.at[cid, pl.ds(r0, rps)])
    if tail:
        @pl.when(sid == 0)
        def _():
            pltpu.sync_copy(acc_sh.at[pl.ds(NS * rps, tail)],
                            out_p_hbm.at[cid, pl.ds(NS * rps, tail)])


def kernel(h, w, edge_index, W, a):
    n, d_in = h.shape
    d_out = W.shape[0]
    e_cnt = edge_index.shape[1]
    n_chunks = e_cnt // CHUNK
    chunks_per_tile = (n_chunks + NW - 1) // NW

    z, s1, s2 = pl.pallas_call(
        _prep_body,
        out_shape=[
            jax.ShapeDtypeStruct((n, d_out), jnp.float32),
            jax.ShapeDtypeStruct((n,), jnp.float32),
            jax.ShapeDtypeStruct((n,), jnp.float32),
        ],
    )(h, W, a.reshape(2, d_out))

    pad = jnp.zeros((2 * NB * CHUNK,), edge_index.dtype)
    src = jnp.concatenate([edge_index[0], pad])
    dst = jnp.concatenate([edge_index[1], pad])

    mesh = plsc.VectorSubcoreMesh(core_axis_name="c", subcore_axis_name="s",
                                  num_cores=NC, num_subcores=NS)
    sc_fn = functools.partial(
        pl.kernel,
        out_type=[
            jax.ShapeDtypeStruct((NC, n, d_out), jnp.float32),
            jax.ShapeDtypeStruct((NW, 1, n), jnp.float32),
        ],
        mesh=mesh,
        scratch_types=[
            pltpu.VMEM((n,), jnp.float32),
            pltpu.VMEM((n,), jnp.float32),
            pltpu.VMEM((n,), jnp.float32),
            [pltpu.VMEM((NB * CHUNK,), jnp.int32) for _ in range(2)],
            [pltpu.VMEM((NB * CHUNK,), jnp.int32) for _ in range(2)],
            [pltpu.VMEM((CHUNK,), jnp.int32) for _ in range(NR)],
            [pltpu.VMEM((CHUNK, d_out), jnp.float32) for _ in range(NR)],
            pltpu.VMEM_SHARED((n, d_out), jnp.float32),
            [pltpu.SemaphoreType.DMA for _ in range(2)],
            [pltpu.SemaphoreType.DMA for _ in range(NR)],
            [pltpu.SemaphoreType.DMA for _ in range(NR)],
        ],
        compiler_params=pltpu.CompilerParams(needs_layout_passes=False),
    )(functools.partial(_sc_body, n, n_chunks, chunks_per_tile, e_cnt))
    partials, denoms = sc_fn(z, src, dst, s1, s2)

    h_out = pl.pallas_call(
        _combine_body,
        out_shape=jax.ShapeDtypeStruct((n, d_out), jnp.float32),
    )(partials, denoms.reshape(NW, n))
    return h_out
